# baseline (speedup 1.0000x reference)
import dataclasses

import jax
import jax.numpy as jnp
from jax import lax
from jax.experimental import pallas as pl
from jax.experimental.pallas import tpu as pltpu
from jax.experimental.pallas import tpu_sc as plsc

_ROWS, _COLS = 128, 32768
_NSUB = 32
_RPW = _ROWS // _NSUB


def _count_body(x_hbm, cnt_hbm, buf0, buf1, acc_v, sem0, sem1, semc):
    wid = lax.axis_index("s") * 2 + lax.axis_index("c")
    base = wid * _RPW
    bufs = (buf0, buf1)
    sems = (sem0, sem1)

    cps = [None] * _RPW
    cps[0] = pltpu.async_copy(x_hbm.at[base], buf0, sem0)
    cps[1] = pltpu.async_copy(x_hbm.at[base + 1], buf1, sem1)

    lane = lax.iota(jnp.int32, 16)
    cnt_cps = []
    for r in range(_RPW):
        buf = bufs[r % 2]
        cps[r].wait()
        v0 = buf[pl.ds(0, 16)]
        pivot = jnp.max(jnp.where(lane == 0, v0, -jnp.inf))

        @plsc.parallel_loop(0, _COLS, step=16, unroll=8,
                            carry=jnp.zeros((16,), jnp.int32))
        def acc_loop(i, acc, buf=buf, pivot=pivot):
            v = buf[pl.ds(i, 16)]
            return acc + jnp.where(v > pivot, 1, 0).astype(jnp.int32)

        acc_v[r] = acc_loop
        cnt_cps.append(pltpu.async_copy(acc_v.at[r], cnt_hbm.at[base + r], semc))
        if r + 2 < _RPW:
            cps[r + 2] = pltpu.async_copy(
                x_hbm.at[base + r + 2], bufs[r % 2], sems[r % 2]
            )

    for cp in cnt_cps:
        cp.wait()


def _sc_compiler_params():
    cp = pltpu.CompilerParams()
    if "needs_layout_passes" in pltpu.CompilerParams.__dataclass_fields__:
        cp = dataclasses.replace(cp, needs_layout_passes=False)
    return cp


def _sc_counts(scores):
    mesh = plsc.VectorSubcoreMesh(core_axis_name="c", subcore_axis_name="s")
    k = pl.kernel(
        _count_body,
        compiler_params=_sc_compiler_params(),
        out_type=jax.ShapeDtypeStruct((_ROWS, 16), jnp.int32),
        mesh=mesh,
        scratch_types=[
            pltpu.VMEM((_COLS,), jnp.float32),
            pltpu.VMEM((_COLS,), jnp.float32),
            pltpu.VMEM((_RPW, 16), jnp.int32),
            pltpu.SemaphoreType.DMA,
            pltpu.SemaphoreType.DMA,
            pltpu.SemaphoreType.DMA,
        ],
    )
    return k(scores)


def kernel(scores):
    cnt16 = _sc_counts(scores)
    cnt = jnp.sum(cnt16, axis=1)
    return jnp.zeros((_ROWS, _COLS), jnp.float32).at[jnp.arange(_ROWS), cnt].set(1.0)


# speedup vs baseline: 1.6793x; 1.6793x over previous
import dataclasses

import jax
import jax.numpy as jnp
from jax import lax
from jax.experimental import pallas as pl
from jax.experimental.pallas import tpu as pltpu
from jax.experimental.pallas import tpu_sc as plsc

_ROWS, _COLS = 128, 32768
_NSUB = 32
_K_SC = 64
_RPW = _K_SC // _NSUB
_TC_BLK = 32


def _count_body(x_hbm, cnt_hbm, buf0, buf1, acc_v, sem0, sem1, semc):
    wid = lax.axis_index("s") * 2 + lax.axis_index("c")
    base = wid * _RPW
    bufs = (buf0, buf1)
    sems = (sem0, sem1)

    cps = [None] * _RPW
    cps[0] = pltpu.async_copy(x_hbm.at[base], buf0, sem0)
    if _RPW > 1:
        cps[1] = pltpu.async_copy(x_hbm.at[base + 1], buf1, sem1)

    lane = lax.iota(jnp.int32, 16)
    cnt_cps = []
    for r in range(_RPW):
        buf = bufs[r % 2]
        cps[r].wait()
        v0 = buf[pl.ds(0, 16)]
        pivot = jnp.max(jnp.where(lane == 0, v0, -jnp.inf))

        @plsc.parallel_loop(0, _COLS, step=16, unroll=8,
                            carry=jnp.zeros((16,), jnp.int32))
        def acc_loop(i, acc, buf=buf, pivot=pivot):
            v = buf[pl.ds(i, 16)]
            return acc + jnp.where(v > pivot, 1, 0).astype(jnp.int32)

        acc_v[r] = acc_loop
        cnt_cps.append(pltpu.async_copy(acc_v.at[r], cnt_hbm.at[base + r], semc))
        if r + 2 < _RPW:
            cps[r + 2] = pltpu.async_copy(
                x_hbm.at[base + r + 2], bufs[r % 2], sems[r % 2]
            )

    for cp in cnt_cps:
        cp.wait()


def _sc_compiler_params():
    cp = pltpu.CompilerParams()
    if "needs_layout_passes" in pltpu.CompilerParams.__dataclass_fields__:
        cp = dataclasses.replace(cp, needs_layout_passes=False)
    return cp


def _sc_counts(scores):
    mesh = plsc.VectorSubcoreMesh(core_axis_name="c", subcore_axis_name="s")
    k = pl.kernel(
        _count_body,
        out_type=jax.ShapeDtypeStruct((_K_SC, 16), jnp.int32),
        mesh=mesh,
        compiler_params=_sc_compiler_params(),
        scratch_types=[
            pltpu.VMEM((_COLS,), jnp.float32),
            pltpu.VMEM((_COLS,), jnp.float32),
            pltpu.VMEM((_RPW, 16), jnp.int32),
            pltpu.SemaphoreType.DMA,
            pltpu.SemaphoreType.DMA,
            pltpu.SemaphoreType.DMA,
        ],
    )
    return k(scores)


def _fused_body(x_ref, o_ref):
    x = x_ref[...]
    pivot = x[:, 0:1]
    gt = (x > pivot).astype(jnp.int32)
    cnt = jnp.sum(gt, axis=1, keepdims=True)
    iota = lax.broadcasted_iota(jnp.int32, x.shape, 1)
    o_ref[...] = (iota == cnt).astype(jnp.float32)


def _onehot_from_counts_body(c16_ref, buf_ref, o_ref):
    del buf_ref
    cnt = jnp.sum(c16_ref[...], axis=1, keepdims=True)
    iota = lax.broadcasted_iota(jnp.int32, (_TC_BLK, _COLS), 1)
    o_ref[...] = (iota == cnt).astype(jnp.float32)


def kernel(scores):
    cnt16 = _sc_counts(scores)

    n_tc = (_ROWS - _K_SC) // _TC_BLK
    part = pl.pallas_call(
        _fused_body,
        grid=(n_tc,),
        in_specs=[pl.BlockSpec((_TC_BLK, _COLS),
                               lambda i: (i + _K_SC // _TC_BLK, 0))],
        out_specs=pl.BlockSpec((_TC_BLK, _COLS),
                               lambda i: (i + _K_SC // _TC_BLK, 0)),
        out_shape=jax.ShapeDtypeStruct((_ROWS, _COLS), jnp.float32),
    )(scores)

    out = pl.pallas_call(
        _onehot_from_counts_body,
        grid=(_K_SC // _TC_BLK,),
        in_specs=[
            pl.BlockSpec((_TC_BLK, 16), lambda i: (i, 0)),
            pl.BlockSpec(memory_space=pltpu.MemorySpace.HBM),
        ],
        out_specs=pl.BlockSpec((_TC_BLK, _COLS), lambda i: (i, 0)),
        out_shape=jax.ShapeDtypeStruct((_ROWS, _COLS), jnp.float32),
        input_output_aliases={1: 0},
    )(cnt16, part)
    return out


# speedup vs baseline: 4.8525x; 2.8896x over previous
import jax
import jax.numpy as jnp
from jax.experimental import pallas as pl

_ROWS, _COLS = 128, 32768
_BLOCK_ROWS = 64


def _onehot_rank_body(x_ref, o_ref):
    x = x_ref[...]
    pivot = x[:, 0:1]
    gt = (x > pivot).astype(jnp.int32)
    cnt = jnp.sum(gt, axis=1, keepdims=True)
    iota = jax.lax.broadcasted_iota(jnp.int32, x.shape, 1)
    o_ref[...] = (iota == cnt).astype(jnp.float32)


def kernel(scores):
    return pl.pallas_call(
        _onehot_rank_body,
        grid=(_ROWS // _BLOCK_ROWS,),
        in_specs=[pl.BlockSpec((_BLOCK_ROWS, _COLS), lambda i: (i, 0))],
        out_specs=pl.BlockSpec((_BLOCK_ROWS, _COLS), lambda i: (i, 0)),
        out_shape=jax.ShapeDtypeStruct((_ROWS, _COLS), jnp.float32),
    )(scores)
